# Initial kernel scaffold; baseline (speedup 1.0000x reference)
import jax
import jax.numpy as jnp
from jax.experimental import pallas as pl


def kernel(x, edge_index, W1l, b1l, W1r, W2l, b2l, W2r, W3l, b3l, W3r):
    raise NotImplementedError("write your pallas kernel here")


# speedup vs baseline: 4.4088x; 4.4088x over previous
import dataclasses
import functools

import jax
import jax.numpy as jnp
from jax import lax
from jax.experimental import pallas as pl
from jax.experimental.pallas import tpu as pltpu
from jax.experimental.pallas import tpu_sc as plsc

N = 10000
E = 320000
D = 128

NC = 2
NS = 16
LANES = 128
ROWS_PER_WORKER = (E + NC * NS * LANES - 1) // (NC * NS * LANES)
E_ROWS = NC * NS * ROWS_PER_WORKER
N_PAD = NS * 640
CROWS = N_PAD // D

_MESH = plsc.VectorSubcoreMesh(core_axis_name="c", subcore_axis_name="s",
                               num_cores=NC, num_subcores=NS)


def _segsum_body(with_counts, h_hbm, src_hbm, dst_hbm, *refs):
    if with_counts:
        (out_hbm, cnt_hbm, srcv, dstv, rows, cntv, idxv, acc, cacc,
         sem) = refs
    else:
        out_hbm, srcv, dstv, rows, acc, sem = refs
    c = lax.axis_index("c")
    s = lax.axis_index("s")
    zv = jnp.zeros((16,), jnp.float32)

    @pl.loop(0, LANES)
    def _(i):
        @pl.loop(0, D, step=16)
        def _(j):
            rows[i, pl.ds(j, 16)] = zv

    @pl.loop(0, 5)
    def _(k):
        pltpu.sync_copy(rows, acc.at[pl.ds(s * 640 + k * LANES, LANES)])

    if with_counts:
        @pl.loop(0, CROWS)
        def _(i):
            @pl.loop(0, D, step=16)
            def _(j):
                cntv[i, pl.ds(j, 16)] = zv

        @pl.when(s < 10)
        def _():
            pltpu.sync_copy(rows.at[pl.ds(0, 8)], cacc.at[pl.ds(s * 8, 8)])

        @pl.loop(0, CROWS, step=16)
        def _(k):
            idxv[pl.ds(k, 16)] = lax.iota(jnp.int32, 16) + k

    plsc.subcore_barrier()

    base_row = (c * NS + s) * ROWS_PER_WORKER
    ones16 = jnp.ones((16,), jnp.float32)

    @pl.loop(0, ROWS_PER_WORKER)
    def _(j):
        r = base_row + j
        pltpu.sync_copy(src_hbm.at[r], srcv)
        pltpu.sync_copy(dst_hbm.at[r], dstv)
        pltpu.async_copy(h_hbm.at[srcv], rows, sem).wait()
        pltpu.sync_copy(rows, acc.at[dstv], add=True)
        if with_counts:
            @pl.loop(0, LANES, step=16)
            def _(t):
                d16 = dstv[pl.ds(t, 16)]
                plsc.addupdate_scatter(
                    cntv, [lax.shift_right_logical(d16, 7),
                           lax.bitwise_and(d16, 127)], ones16)

    if with_counts:
        pltpu.sync_copy(cntv, cacc.at[idxv], add=True)

    plsc.subcore_barrier()

    pltpu.sync_copy(acc.at[pl.ds(s * 640, 640)],
                    out_hbm.at[c, pl.ds(s * 640, 640)])
    if with_counts:
        @pl.when(s < 10)
        def _():
            pltpu.sync_copy(cacc.at[pl.ds(s * 8, 8)],
                            cnt_hbm.at[c, pl.ds(s * 8, 8)])


def _make_segsum(with_counts):
    out_type = [jax.ShapeDtypeStruct((NC, N_PAD, D), jnp.float32)]
    scratch = [
        pltpu.VMEM((LANES,), jnp.int32),
        pltpu.VMEM((LANES,), jnp.int32),
        pltpu.VMEM((LANES, D), jnp.float32),
    ]
    if with_counts:
        out_type.append(jax.ShapeDtypeStruct((NC, CROWS, D), jnp.float32))
        scratch += [
            pltpu.VMEM((CROWS, D), jnp.float32),
            pltpu.VMEM((CROWS,), jnp.int32),
        ]
    scratch.append(pltpu.VMEM_SHARED((N_PAD, D), jnp.float32))
    if with_counts:
        scratch.append(pltpu.VMEM_SHARED((CROWS, D), jnp.float32))
    scratch.append(pltpu.SemaphoreType.DMA)
    cp = pltpu.CompilerParams()
    if "needs_layout_passes" in pltpu.CompilerParams.__dataclass_fields__:
        cp = dataclasses.replace(cp, needs_layout_passes=False)
    return pl.kernel(
        functools.partial(_segsum_body, with_counts),
        out_type=tuple(out_type) if with_counts else out_type[0],
        mesh=_MESH,
        scratch_types=scratch,
        compiler_params=cp,
        name="sc_segsum",
    )


_segsum_with_counts = _make_segsum(True)
_segsum = _make_segsum(False)

_BLK = 1024


def _root_body(h_ref, w_ref, o_ref):
    o_ref[...] = lax.dot_general(
        h_ref[...], w_ref[...], (((1,), (1,)), ((), ())),
        preferred_element_type=jnp.float32,
        precision=lax.Precision.HIGHEST)


def _root(h, w):
    return pl.pallas_call(
        _root_body,
        grid=(pl.cdiv(N, _BLK),),
        in_specs=[
            pl.BlockSpec((_BLK, D), lambda i: (i, 0)),
            pl.BlockSpec((D, D), lambda i: (0, 0)),
        ],
        out_specs=pl.BlockSpec((_BLK, D), lambda i: (i, 0)),
        out_shape=jax.ShapeDtypeStruct((N, D), jnp.float32),
    )(h, w)


def _combine_body(relu, p_ref, c_ref, hr_ref, w_ref, b_ref, o_ref):
    summed = p_ref[0] + p_ref[1]
    mean = summed / jnp.maximum(c_ref[...], 1.0)
    out = lax.dot_general(
        mean, w_ref[...], (((1,), (1,)), ((), ())),
        preferred_element_type=jnp.float32,
        precision=lax.Precision.HIGHEST)
    out = out + b_ref[...] + hr_ref[...]
    if relu:
        out = jnp.maximum(out, 0.0)
    o_ref[...] = out


def _combine(p, cnt, hr, wl, bl, relu):
    return pl.pallas_call(
        functools.partial(_combine_body, relu),
        grid=(pl.cdiv(N, _BLK),),
        in_specs=[
            pl.BlockSpec((NC, _BLK, D), lambda i: (0, i, 0)),
            pl.BlockSpec((_BLK, 1), lambda i: (i, 0)),
            pl.BlockSpec((_BLK, D), lambda i: (i, 0)),
            pl.BlockSpec((D, D), lambda i: (0, 0)),
            pl.BlockSpec((1, D), lambda i: (0, 0)),
        ],
        out_specs=pl.BlockSpec((_BLK, D), lambda i: (i, 0)),
        out_shape=jax.ShapeDtypeStruct((N, D), jnp.float32),
    )(p, cnt, hr, wl, bl.reshape(1, D))


def kernel(x, edge_index, W1l, b1l, W1r, W2l, b2l, W2r, W3l, b3l, W3r):
    pad = E_ROWS * LANES - E
    src = jnp.concatenate([edge_index[0], jnp.zeros((pad,), jnp.int32)])
    dst = jnp.concatenate([edge_index[1], jnp.full((pad,), N, jnp.int32)])
    srcp = src.reshape(E_ROWS, LANES)
    dstp = dst.reshape(E_ROWS, LANES)

    hr1 = _root(x, W1r)
    p1, cnt_grid = _segsum_with_counts(x, srcp, dstp)
    cnt = (cnt_grid[0] + cnt_grid[1]).reshape(N_PAD, 1)
    h1 = _combine(p1, cnt, hr1, W1l, b1l, relu=True)

    hr2 = _root(h1, W2r)
    p2 = _segsum(h1, srcp, dstp)
    h2 = _combine(p2, cnt, hr2, W2l, b2l, relu=True)

    hr3 = _root(h2, W3r)
    p3 = _segsum(h2, srcp, dstp)
    return _combine(p3, cnt, hr3, W3l, b3l, relu=False)
